# Initial kernel scaffold; baseline (speedup 1.0000x reference)
import functools

import jax
import jax.numpy as jnp
from jax import lax
from jax.experimental import pallas as pl
from jax.experimental.pallas import tpu as pltpu
from jax.experimental.pallas import tpu_sc as plsc

N = 10000
E = 320000
HID = 128
NC, NS = 2, 16
NW = NC * NS
NPAD = 10240
ECH = 128
NECH = 80
EPAD = NW * NECH * ECH
BCH = 64
NBCH = (NPAD // NW) // BCH
RPT = NPAD // NS

_mesh = plsc.VectorSubcoreMesh(core_axis_name="c", subcore_axis_name="s")


@functools.partial(
    pl.kernel,
    out_type=(
        jax.ShapeDtypeStruct((NPAD, HID), jnp.float32),
        jax.ShapeDtypeStruct((NC, 2, NPAD), jnp.float32),
    ),
    mesh=_mesh,
    scratch_types=[
        pltpu.VMEM((BCH,), jnp.int32),
        pltpu.VMEM((BCH, HID), jnp.float32),
        pltpu.VMEM((NECH, ECH), jnp.int32),
        pltpu.VMEM((NECH, ECH), jnp.int32),
        pltpu.VMEM((ECH,), jnp.float32),
        pltpu.VMEM_SHARED((NPAD,), jnp.float32),
        pltpu.VMEM_SHARED((NPAD,), jnp.float32),
        pltpu.SemaphoreType.DMA,
    ],
)
def _phase_a(emb_hbm, batch3, src3, dst3, ones_hbm, zdeg_hbm,
             embs_out, degp_out,
             bidx_v, erows_v, sidx_v, didx_v, ones_v, degs_sh, degd_sh, sem):
    cid = lax.axis_index("c")
    sid = lax.axis_index("s")
    wid = sid * NC + cid
    r0 = sid * RPT
    pltpu.sync_copy(zdeg_hbm.at[pl.ds(r0, RPT)], degs_sh.at[pl.ds(r0, RPT)])
    pltpu.sync_copy(zdeg_hbm.at[pl.ds(r0, RPT)], degd_sh.at[pl.ds(r0, RPT)])
    pltpu.sync_copy(ones_hbm, ones_v)
    for c in range(NBCH):
        pltpu.sync_copy(batch3.at[wid, c], bidx_v)
        pltpu.async_copy(emb_hbm.at[bidx_v], erows_v, sem).wait()
        pltpu.sync_copy(erows_v,
                        embs_out.at[pl.ds(wid * (NPAD // NW) + c * BCH, BCH)])
    pltpu.sync_copy(src3.at[wid], sidx_v)
    pltpu.sync_copy(dst3.at[wid], didx_v)
    plsc.subcore_barrier()

    def deg_chunk(c, carry):
        pltpu.sync_copy(ones_v, degs_sh.at[sidx_v.at[c]], add=True)
        pltpu.sync_copy(ones_v, degd_sh.at[didx_v.at[c]], add=True)
        return carry

    lax.fori_loop(0, NECH, deg_chunk, 0)
    plsc.subcore_barrier()
    pltpu.sync_copy(degs_sh.at[pl.ds(r0, RPT)],
                    degp_out.at[cid, 0, pl.ds(r0, RPT)])
    pltpu.sync_copy(degd_sh.at[pl.ds(r0, RPT)],
                    degp_out.at[cid, 1, pl.ds(r0, RPT)])


@functools.partial(
    pl.kernel,
    out_type=jax.ShapeDtypeStruct((NC, NPAD, HID), jnp.float32),
    mesh=_mesh,
    scratch_types=[
        pltpu.VMEM((NECH, ECH), jnp.int32),
        pltpu.VMEM((NECH, ECH), jnp.int32),
        pltpu.VMEM((ECH, HID), jnp.float32),
        pltpu.VMEM((ECH, HID), jnp.float32),
        pltpu.VMEM_SHARED((NPAD, HID), jnp.float32),
        pltpu.SemaphoreType.DMA,
        pltpu.SemaphoreType.DMA,
    ],
)
def _msgpass(x_hbm, src3, dst3, zfeat_hbm,
             part_out,
             sidx_v, didx_v, bufa, bufb, acc_sh, sema, semb):
    cid = lax.axis_index("c")
    sid = lax.axis_index("s")
    wid = sid * NC + cid
    r0 = sid * RPT
    pltpu.sync_copy(zfeat_hbm.at[pl.ds(r0, RPT)], acc_sh.at[pl.ds(r0, RPT)])
    pltpu.sync_copy(src3.at[wid], sidx_v)
    pltpu.sync_copy(dst3.at[wid], didx_v)
    plsc.subcore_barrier()

    def body(i, carry):
        c0 = 2 * i
        c1 = c0 + 1
        ca = pltpu.async_copy(x_hbm.at[sidx_v.at[c0]], bufa, sema)
        cb = pltpu.async_copy(x_hbm.at[sidx_v.at[c1]], bufb, semb)
        ca.wait()
        pltpu.sync_copy(bufa, acc_sh.at[didx_v.at[c0]], add=True)
        cb.wait()
        pltpu.sync_copy(bufb, acc_sh.at[didx_v.at[c1]], add=True)
        return carry

    lax.fori_loop(0, NECH // 2, body, 0)
    plsc.subcore_barrier()
    pltpu.sync_copy(acc_sh.at[pl.ds(r0, RPT)],
                    part_out.at[cid, pl.ds(r0, RPT)])


RB = 1024


def _scale_body(x_ref, d0_ref, d1_ref, o_ref):
    deg = d0_ref[...] + d1_ref[...]
    o_ref[...] = x_ref[...] * lax.rsqrt(jnp.clip(deg, 1.0, None))


def _scale(x, d0, d1):
    return pl.pallas_call(
        _scale_body,
        grid=(NPAD // RB,),
        in_specs=[
            pl.BlockSpec((RB, HID), lambda i: (i, 0)),
            pl.BlockSpec((RB, 1), lambda i: (i, 0)),
            pl.BlockSpec((RB, 1), lambda i: (i, 0)),
        ],
        out_specs=pl.BlockSpec((RB, HID), lambda i: (i, 0)),
        out_shape=jax.ShapeDtypeStruct((NPAD, HID), jnp.float32),
    )(x, d0, d1)


def _layer_body(p0_ref, p1_ref, dd0_ref, dd1_ref, ds0_ref, ds1_ref,
                w_ref, b_ref, o_ref, *, post_src):
    agg = p0_ref[...] + p1_ref[...]
    nd = lax.rsqrt(jnp.clip(dd0_ref[...] + dd1_ref[...], 1.0, None))
    y = jnp.dot(agg * nd, w_ref[...], preferred_element_type=jnp.float32)
    y = jnp.maximum(y + b_ref[...], 0.0)
    if post_src:
        y = y * lax.rsqrt(jnp.clip(ds0_ref[...] + ds1_ref[...], 1.0, None))
    o_ref[...] = y


def _layer(p0, p1, dd0, dd1, ds0, ds1, w, b, post_src):
    col = pl.BlockSpec((RB, 1), lambda i: (i, 0))
    return pl.pallas_call(
        functools.partial(_layer_body, post_src=post_src),
        grid=(NPAD // RB,),
        in_specs=[
            pl.BlockSpec((RB, HID), lambda i: (i, 0)),
            pl.BlockSpec((RB, HID), lambda i: (i, 0)),
            col, col, col, col,
            pl.BlockSpec((HID, HID), lambda i: (0, 0)),
            pl.BlockSpec((1, HID), lambda i: (0, 0)),
        ],
        out_specs=pl.BlockSpec((RB, HID), lambda i: (i, 0)),
        out_shape=jax.ShapeDtypeStruct((NPAD, HID), jnp.float32),
    )(p0, p1, dd0, dd1, ds0, ds1, w, b)


def kernel(batch, edge_index, emb_table, W1, b1, W2, b2):
    src = edge_index[0]
    dst = edge_index[1]
    pe = EPAD - E
    pad_rows = N + jnp.arange(pe, dtype=jnp.int32) % (NPAD - N)
    src3 = jnp.concatenate([src, pad_rows]).reshape(NW, NECH, ECH)
    dst3 = jnp.concatenate([dst, pad_rows]).reshape(NW, NECH, ECH)
    pb = NPAD - N
    batch3 = jnp.concatenate(
        [batch, jnp.arange(pb, dtype=jnp.int32) * 131 % emb_table.shape[0]]
    ).reshape(NW, NBCH, BCH)
    ones_e = jnp.ones((ECH,), jnp.float32)
    zdeg = jnp.zeros((NPAD,), jnp.float32)
    zfeat = jnp.zeros((NPAD, HID), jnp.float32)

    embs, degp = _phase_a(emb_table, batch3, src3, dst3, ones_e, zdeg)
    ds0 = degp[0, 0].reshape(NPAD, 1)
    ds1 = degp[1, 0].reshape(NPAD, 1)
    dd0 = degp[0, 1].reshape(NPAD, 1)
    dd1 = degp[1, 1].reshape(NPAD, 1)
    b1r = b1.reshape(1, HID)
    b2r = b2.reshape(1, HID)

    h0 = _scale(embs, ds0, ds1)
    parts = _msgpass(h0, src3, dst3, zfeat)
    h1 = _layer(parts[0], parts[1], dd0, dd1, ds0, ds1, W1, b1r, True)
    parts2 = _msgpass(h1, src3, dst3, zfeat)
    out = _layer(parts2[0], parts2[1], dd0, dd1, ds0, ds1, W2, b2r, False)
    return out[:N]


# speedup vs baseline: 6.7663x; 6.7663x over previous
import functools

import jax
import jax.numpy as jnp
from jax import lax
from jax.experimental import pallas as pl
from jax.experimental.pallas import tpu as pltpu
from jax.experimental.pallas import tpu_sc as plsc

N = 10000
E = 320000
HID = 128
NC, NS = 2, 16
NW = NC * NS
NPAD = 10240
ECH = 128
NECH = 80
EPAD = NW * NECH * ECH
BCH = 64
NBCH = (NPAD // NW) // BCH
RPT = NPAD // NS

_mesh = plsc.VectorSubcoreMesh(core_axis_name="c", subcore_axis_name="s")
_sc_params = pltpu.CompilerParams(use_tc_tiling_on_sc=False)


@functools.partial(
    pl.kernel,
    out_type=(
        jax.ShapeDtypeStruct((NPAD, HID), jnp.float32),
        jax.ShapeDtypeStruct((NC, 2, NPAD), jnp.float32),
    ),
    mesh=_mesh,
    scratch_types=[
        pltpu.VMEM((BCH,), jnp.int32),
        pltpu.VMEM((BCH, HID), jnp.float32),
        pltpu.VMEM((NECH, ECH), jnp.int32),
        pltpu.VMEM((NECH, ECH), jnp.int32),
        pltpu.VMEM((ECH,), jnp.float32),
        pltpu.VMEM_SHARED((NPAD,), jnp.float32),
        pltpu.VMEM_SHARED((NPAD,), jnp.float32),
        pltpu.SemaphoreType.DMA,
    ],
    compiler_params=_sc_params,
)
def _phase_a(emb_hbm, batch3, src3, dst3, ones_hbm, zdeg_hbm,
             embs_out, degp_out,
             bidx_v, erows_v, sidx_v, didx_v, ones_v, degs_sh, degd_sh, sem):
    cid = lax.axis_index("c")
    sid = lax.axis_index("s")
    wid = sid * NC + cid
    r0 = sid * RPT
    pltpu.sync_copy(zdeg_hbm.at[pl.ds(r0, RPT)], degs_sh.at[pl.ds(r0, RPT)])
    pltpu.sync_copy(zdeg_hbm.at[pl.ds(r0, RPT)], degd_sh.at[pl.ds(r0, RPT)])
    pltpu.sync_copy(ones_hbm, ones_v)
    for c in range(NBCH):
        pltpu.sync_copy(batch3.at[wid, c], bidx_v)
        pltpu.async_copy(emb_hbm.at[bidx_v], erows_v, sem).wait()
        pltpu.sync_copy(erows_v,
                        embs_out.at[pl.ds(wid * (NPAD // NW) + c * BCH, BCH)])
    pltpu.sync_copy(src3.at[wid], sidx_v)
    pltpu.sync_copy(dst3.at[wid], didx_v)
    plsc.subcore_barrier()

    def deg_chunk(c, carry):
        pltpu.sync_copy(ones_v, degs_sh.at[sidx_v.at[c]], add=True)
        pltpu.sync_copy(ones_v, degd_sh.at[didx_v.at[c]], add=True)
        return carry

    lax.fori_loop(0, NECH, deg_chunk, 0)
    plsc.subcore_barrier()
    pltpu.sync_copy(degs_sh.at[pl.ds(r0, RPT)],
                    degp_out.at[cid, 0, pl.ds(r0, RPT)])
    pltpu.sync_copy(degd_sh.at[pl.ds(r0, RPT)],
                    degp_out.at[cid, 1, pl.ds(r0, RPT)])


@functools.partial(
    pl.kernel,
    out_type=jax.ShapeDtypeStruct((NC, NPAD, HID), jnp.float32),
    mesh=_mesh,
    scratch_types=[
        pltpu.VMEM((2, ECH), jnp.int32),
        pltpu.VMEM((2, ECH), jnp.int32),
        pltpu.VMEM((ECH, HID), jnp.float32),
        pltpu.VMEM((ECH, HID), jnp.float32),
        pltpu.VMEM_SHARED((NPAD, HID), jnp.float32),
        pltpu.SemaphoreType.DMA,
        pltpu.SemaphoreType.DMA,
    ],
    compiler_params=_sc_params,
)
def _msgpass(x_hbm, src3, dst3, zfeat_hbm,
             part_out,
             csrc, cdst, bufa, bufb, acc_sh, sema, semb):
    cid = lax.axis_index("c")
    sid = lax.axis_index("s")
    wid = sid * NC + cid
    r0 = sid * RPT
    pltpu.sync_copy(zfeat_hbm.at[pl.ds(r0, RPT)], acc_sh.at[pl.ds(r0, RPT)])
    plsc.subcore_barrier()

    def body(i, carry):
        c0 = 2 * i
        c1 = c0 + 1
        pltpu.sync_copy(src3.at[wid, c0], csrc.at[0])
        pltpu.sync_copy(src3.at[wid, c1], csrc.at[1])
        pltpu.sync_copy(dst3.at[wid, c0], cdst.at[0])
        pltpu.sync_copy(dst3.at[wid, c1], cdst.at[1])
        ca = pltpu.async_copy(x_hbm.at[csrc.at[0]], bufa, sema)
        cb = pltpu.async_copy(x_hbm.at[csrc.at[1]], bufb, semb)
        ca.wait()
        pltpu.sync_copy(bufa, acc_sh.at[cdst.at[0]], add=True)
        cb.wait()
        pltpu.sync_copy(bufb, acc_sh.at[cdst.at[1]], add=True)
        return carry

    lax.fori_loop(0, NECH // 2, body, 0)
    plsc.subcore_barrier()
    pltpu.sync_copy(acc_sh.at[pl.ds(r0, RPT)],
                    part_out.at[cid, pl.ds(r0, RPT)])


RB = 1024


def _scale_body(x_ref, d0_ref, d1_ref, o_ref):
    deg = d0_ref[...] + d1_ref[...]
    o_ref[...] = x_ref[...] * lax.rsqrt(jnp.clip(deg, 1.0, None))


def _scale(x, d0, d1):
    return pl.pallas_call(
        _scale_body,
        grid=(NPAD // RB,),
        in_specs=[
            pl.BlockSpec((RB, HID), lambda i: (i, 0)),
            pl.BlockSpec((RB, 1), lambda i: (i, 0)),
            pl.BlockSpec((RB, 1), lambda i: (i, 0)),
        ],
        out_specs=pl.BlockSpec((RB, HID), lambda i: (i, 0)),
        out_shape=jax.ShapeDtypeStruct((NPAD, HID), jnp.float32),
    )(x, d0, d1)


def _layer_body(p0_ref, p1_ref, dd0_ref, dd1_ref, ds0_ref, ds1_ref,
                w_ref, b_ref, o_ref, *, post_src):
    agg = p0_ref[...] + p1_ref[...]
    nd = lax.rsqrt(jnp.clip(dd0_ref[...] + dd1_ref[...], 1.0, None))
    y = jnp.dot(agg * nd, w_ref[...], preferred_element_type=jnp.float32)
    y = jnp.maximum(y + b_ref[...], 0.0)
    if post_src:
        y = y * lax.rsqrt(jnp.clip(ds0_ref[...] + ds1_ref[...], 1.0, None))
    o_ref[...] = y


def _layer(p0, p1, dd0, dd1, ds0, ds1, w, b, post_src):
    col = pl.BlockSpec((RB, 1), lambda i: (i, 0))
    return pl.pallas_call(
        functools.partial(_layer_body, post_src=post_src),
        grid=(NPAD // RB,),
        in_specs=[
            pl.BlockSpec((RB, HID), lambda i: (i, 0)),
            pl.BlockSpec((RB, HID), lambda i: (i, 0)),
            col, col, col, col,
            pl.BlockSpec((HID, HID), lambda i: (0, 0)),
            pl.BlockSpec((1, HID), lambda i: (0, 0)),
        ],
        out_specs=pl.BlockSpec((RB, HID), lambda i: (i, 0)),
        out_shape=jax.ShapeDtypeStruct((NPAD, HID), jnp.float32),
    )(p0, p1, dd0, dd1, ds0, ds1, w, b)


def kernel(batch, edge_index, emb_table, W1, b1, W2, b2):
    src = edge_index[0]
    dst = edge_index[1]
    pe = EPAD - E
    pad_rows = N + jnp.arange(pe, dtype=jnp.int32) % (NPAD - N)
    src3 = jnp.concatenate([src, pad_rows]).reshape(NW, NECH, ECH)
    dst3 = jnp.concatenate([dst, pad_rows]).reshape(NW, NECH, ECH)
    pb = NPAD - N
    batch3 = jnp.concatenate(
        [batch, jnp.arange(pb, dtype=jnp.int32) * 131 % emb_table.shape[0]]
    ).reshape(NW, NBCH, BCH)
    ones_e = jnp.ones((ECH,), jnp.float32)
    zdeg = jnp.zeros((NPAD,), jnp.float32)
    zfeat = jnp.zeros((NPAD, HID), jnp.float32)

    embs, degp = _phase_a(emb_table, batch3, src3, dst3, ones_e, zdeg)
    ds0 = degp[0, 0].reshape(NPAD, 1)
    ds1 = degp[1, 0].reshape(NPAD, 1)
    dd0 = degp[0, 1].reshape(NPAD, 1)
    dd1 = degp[1, 1].reshape(NPAD, 1)
    b1r = b1.reshape(1, HID)
    b2r = b2.reshape(1, HID)

    h0 = _scale(embs, ds0, ds1)
    parts = _msgpass(h0, src3, dst3, zfeat)
    h1 = _layer(parts[0], parts[1], dd0, dd1, ds0, ds1, W1, b1r, True)
    parts2 = _msgpass(h1, src3, dst3, zfeat)
    out = _layer(parts2[0], parts2[1], dd0, dd1, ds0, ds1, W2, b2r, False)
    return out[:N]


# speedup vs baseline: 9.3471x; 1.3814x over previous
import functools

import jax
import jax.numpy as jnp
from jax import lax
from jax.experimental import pallas as pl
from jax.experimental.pallas import tpu as pltpu
from jax.experimental.pallas import tpu_sc as plsc

N = 10000
E = 320000
HID = 128
NC, NS = 2, 16
NW = NC * NS
NPAD = 10240
ECH = 128
NECH = 80
EPAD = NW * NECH * ECH
BCH = 64
NBCH = (NPAD // NW) // BCH
RPT = NPAD // NS

_mesh = plsc.VectorSubcoreMesh(core_axis_name="c", subcore_axis_name="s")
_sc_params = pltpu.CompilerParams(use_tc_tiling_on_sc=False)


@functools.partial(
    pl.kernel,
    out_type=(
        jax.ShapeDtypeStruct((NPAD, HID), jnp.float32),
        jax.ShapeDtypeStruct((NC, 2, NPAD), jnp.float32),
    ),
    mesh=_mesh,
    scratch_types=[
        pltpu.VMEM((BCH,), jnp.int32),
        pltpu.VMEM((BCH, HID), jnp.float32),
        pltpu.VMEM((NECH, ECH), jnp.int32),
        pltpu.VMEM((NECH, ECH), jnp.int32),
        pltpu.VMEM((ECH,), jnp.float32),
        pltpu.VMEM_SHARED((NPAD,), jnp.float32),
        pltpu.VMEM_SHARED((NPAD,), jnp.float32),
        pltpu.SemaphoreType.DMA,
    ],
    compiler_params=_sc_params,
)
def _phase_a(emb_hbm, batch3, src3, dst3, ones_hbm, zdeg_hbm,
             embs_out, degp_out,
             bidx_v, erows_v, sidx_v, didx_v, ones_v, degs_sh, degd_sh, sem):
    cid = lax.axis_index("c")
    sid = lax.axis_index("s")
    wid = sid * NC + cid
    r0 = sid * RPT
    pltpu.sync_copy(zdeg_hbm.at[pl.ds(r0, RPT)], degs_sh.at[pl.ds(r0, RPT)])
    pltpu.sync_copy(zdeg_hbm.at[pl.ds(r0, RPT)], degd_sh.at[pl.ds(r0, RPT)])
    pltpu.sync_copy(ones_hbm, ones_v)
    for c in range(NBCH):
        pltpu.sync_copy(batch3.at[wid, c], bidx_v)
        pltpu.async_copy(emb_hbm.at[bidx_v], erows_v, sem).wait()
        pltpu.sync_copy(erows_v,
                        embs_out.at[pl.ds(wid * (NPAD // NW) + c * BCH, BCH)])
    pltpu.sync_copy(src3.at[wid], sidx_v)
    pltpu.sync_copy(dst3.at[wid], didx_v)
    plsc.subcore_barrier()

    def deg_chunk(c, carry):
        pltpu.sync_copy(ones_v, degs_sh.at[sidx_v.at[c]], add=True)
        pltpu.sync_copy(ones_v, degd_sh.at[didx_v.at[c]], add=True)
        return carry

    lax.fori_loop(0, NECH, deg_chunk, 0)
    plsc.subcore_barrier()
    pltpu.sync_copy(degs_sh.at[pl.ds(r0, RPT)],
                    degp_out.at[cid, 0, pl.ds(r0, RPT)])
    pltpu.sync_copy(degd_sh.at[pl.ds(r0, RPT)],
                    degp_out.at[cid, 1, pl.ds(r0, RPT)])


HECH = NECH // 2


@functools.partial(
    pl.kernel,
    out_type=jax.ShapeDtypeStruct((NC, NPAD, HID), jnp.float32),
    mesh=_mesh,
    scratch_types=[
        pltpu.VMEM((HECH, 2, ECH), jnp.int32),
        pltpu.VMEM((ECH, HID), jnp.float32),
        pltpu.VMEM((ECH, HID), jnp.float32),
        pltpu.VMEM_SHARED((NPAD, HID), jnp.float32),
        pltpu.SemaphoreType.DMA,
        pltpu.SemaphoreType.DMA,
        pltpu.SemaphoreType.DMA,
        pltpu.SemaphoreType.DMA,
    ],
    compiler_params=_sc_params,
)
def _msgpass(x_hbm, eidx5, zfeat_hbm,
             part_out,
             eidx_v, bufa, bufb, acc_sh, ga, gb, sa, sb):
    cid = lax.axis_index("c")
    sid = lax.axis_index("s")
    wid = sid * NC + cid
    r0 = sid * RPT
    pltpu.sync_copy(zfeat_hbm.at[pl.ds(r0, RPT)], acc_sh.at[pl.ds(r0, RPT)])
    plsc.subcore_barrier()

    def _gather(c, buf, sem):
        return pltpu.async_copy(x_hbm.at[eidx_v.at[c, 0]], buf, sem)

    def _scatter(c, buf, sem):
        return pltpu.async_copy(buf, acc_sh.at[eidx_v.at[c, 1]], sem, add=True)

    def _gather_wait(c, buf, sem):
        pltpu.make_async_copy(x_hbm.at[eidx_v.at[c, 0]], buf, sem).wait()

    def _scatter_wait(c, buf, sem):
        pltpu.make_async_copy(buf, acc_sh.at[eidx_v.at[c, 1]], sem).wait()

    for h in range(2):
        pltpu.sync_copy(eidx5.at[wid, h], eidx_v)
        _gather(0, bufa, ga)
        _gather(1, bufb, gb)

        def body(k, carry):
            c0 = 2 * k
            c1 = c0 + 1
            _gather_wait(c0, bufa, ga)
            _scatter(c0, bufa, sa)
            _gather_wait(c1, bufb, gb)
            _scatter(c1, bufb, sb)
            _scatter_wait(c0, bufa, sa)
            _gather(c0 + 2, bufa, ga)
            _scatter_wait(c1, bufb, sb)
            _gather(c1 + 2, bufb, gb)
            return carry

        lax.fori_loop(0, HECH // 2 - 1, body, 0)
        _gather_wait(HECH - 2, bufa, ga)
        _scatter(HECH - 2, bufa, sa)
        _gather_wait(HECH - 1, bufb, gb)
        _scatter(HECH - 1, bufb, sb)
        _scatter_wait(HECH - 2, bufa, sa)
        _scatter_wait(HECH - 1, bufb, sb)
    plsc.subcore_barrier()
    pltpu.sync_copy(acc_sh.at[pl.ds(r0, RPT)],
                    part_out.at[cid, pl.ds(r0, RPT)])


RB = 1024


def _scale_body(x_ref, d0_ref, d1_ref, o_ref):
    deg = d0_ref[...] + d1_ref[...]
    o_ref[...] = x_ref[...] * lax.rsqrt(jnp.clip(deg, 1.0, None))


def _scale(x, d0, d1):
    return pl.pallas_call(
        _scale_body,
        grid=(NPAD // RB,),
        in_specs=[
            pl.BlockSpec((RB, HID), lambda i: (i, 0)),
            pl.BlockSpec((RB, 1), lambda i: (i, 0)),
            pl.BlockSpec((RB, 1), lambda i: (i, 0)),
        ],
        out_specs=pl.BlockSpec((RB, HID), lambda i: (i, 0)),
        out_shape=jax.ShapeDtypeStruct((NPAD, HID), jnp.float32),
    )(x, d0, d1)


def _layer_body(p0_ref, p1_ref, dd0_ref, dd1_ref, ds0_ref, ds1_ref,
                w_ref, b_ref, o_ref, *, post_src):
    agg = p0_ref[...] + p1_ref[...]
    nd = lax.rsqrt(jnp.clip(dd0_ref[...] + dd1_ref[...], 1.0, None))
    y = jnp.dot(agg * nd, w_ref[...], preferred_element_type=jnp.float32)
    y = jnp.maximum(y + b_ref[...], 0.0)
    if post_src:
        y = y * lax.rsqrt(jnp.clip(ds0_ref[...] + ds1_ref[...], 1.0, None))
    o_ref[...] = y


def _layer(p0, p1, dd0, dd1, ds0, ds1, w, b, post_src):
    col = pl.BlockSpec((RB, 1), lambda i: (i, 0))
    return pl.pallas_call(
        functools.partial(_layer_body, post_src=post_src),
        grid=(NPAD // RB,),
        in_specs=[
            pl.BlockSpec((RB, HID), lambda i: (i, 0)),
            pl.BlockSpec((RB, HID), lambda i: (i, 0)),
            col, col, col, col,
            pl.BlockSpec((HID, HID), lambda i: (0, 0)),
            pl.BlockSpec((1, HID), lambda i: (0, 0)),
        ],
        out_specs=pl.BlockSpec((RB, HID), lambda i: (i, 0)),
        out_shape=jax.ShapeDtypeStruct((NPAD, HID), jnp.float32),
    )(p0, p1, dd0, dd1, ds0, ds1, w, b)


def kernel(batch, edge_index, emb_table, W1, b1, W2, b2):
    src = edge_index[0]
    dst = edge_index[1]
    pe = EPAD - E
    pad_rows = N + jnp.arange(pe, dtype=jnp.int32) % (NPAD - N)
    src3 = jnp.concatenate([src, pad_rows]).reshape(NW, NECH, ECH)
    dst3 = jnp.concatenate([dst, pad_rows]).reshape(NW, NECH, ECH)
    eidx5 = jnp.stack(
        [src3.reshape(NW, 2, HECH, ECH), dst3.reshape(NW, 2, HECH, ECH)],
        axis=3)
    pb = NPAD - N
    batch3 = jnp.concatenate(
        [batch, jnp.arange(pb, dtype=jnp.int32) * 131 % emb_table.shape[0]]
    ).reshape(NW, NBCH, BCH)
    ones_e = jnp.ones((ECH,), jnp.float32)
    zdeg = jnp.zeros((NPAD,), jnp.float32)
    zfeat = jnp.zeros((NPAD, HID), jnp.float32)

    embs, degp = _phase_a(emb_table, batch3, src3, dst3, ones_e, zdeg)
    ds0 = degp[0, 0].reshape(NPAD, 1)
    ds1 = degp[1, 0].reshape(NPAD, 1)
    dd0 = degp[0, 1].reshape(NPAD, 1)
    dd1 = degp[1, 1].reshape(NPAD, 1)
    b1r = b1.reshape(1, HID)
    b2r = b2.reshape(1, HID)

    h0 = _scale(embs, ds0, ds1)
    parts = _msgpass(h0, eidx5, zfeat)
    h1 = _layer(parts[0], parts[1], dd0, dd1, ds0, ds1, W1, b1r, True)
    parts2 = _msgpass(h1, eidx5, zfeat)
    out = _layer(parts2[0], parts2[1], dd0, dd1, ds0, ds1, W2, b2r, False)
    return out[:N]


# speedup vs baseline: 12.2763x; 1.3134x over previous
import functools

import jax
import jax.numpy as jnp
from jax import lax
from jax.experimental import pallas as pl
from jax.experimental.pallas import tpu as pltpu
from jax.experimental.pallas import tpu_sc as plsc

N = 10000
E = 320000
HID = 128
NC, NS = 2, 16
NW = NC * NS
NPAD = 10240
ECH = 128
EPAD = 327680
NECH = EPAD // NW // ECH
BCH = 64
NBCH = (NPAD // NW) // BCH
RPT = NPAD // NS

_mesh = plsc.VectorSubcoreMesh(core_axis_name="c", subcore_axis_name="s")
_sc_params = pltpu.CompilerParams(use_tc_tiling_on_sc=False)


@functools.partial(
    pl.kernel,
    out_type=(
        jax.ShapeDtypeStruct((NPAD, HID), jnp.float32),
        jax.ShapeDtypeStruct((NC, 2, NPAD), jnp.float32),
    ),
    mesh=_mesh,
    scratch_types=[
        pltpu.VMEM((NBCH, BCH), jnp.int32),
        pltpu.VMEM((NBCH * BCH, HID), jnp.float32),
        pltpu.VMEM((NECH, ECH), jnp.int32),
        pltpu.VMEM((NECH, ECH), jnp.int32),
        pltpu.VMEM((ECH,), jnp.float32),
        pltpu.VMEM_SHARED((NPAD,), jnp.float32),
        pltpu.VMEM_SHARED((NPAD,), jnp.float32),
        pltpu.SemaphoreType.DMA,
        pltpu.SemaphoreType.DMA,
        pltpu.SemaphoreType.DMA,
    ],
    compiler_params=_sc_params,
)
def _phase_a(emb_hbm, batch3, e4, ones_hbm, zdeg_hbm,
             embs_out, degp_out,
             bidx_v, erows_v, sidx_v, didx_v, ones_v, degs_sh, degd_sh,
             gsem, dsem_s, dsem_d):
    cid = lax.axis_index("c")
    sid = lax.axis_index("s")
    wid = sid * NC + cid
    r0 = sid * RPT
    pltpu.sync_copy(zdeg_hbm.at[pl.ds(r0, RPT)], degs_sh.at[pl.ds(r0, RPT)])
    pltpu.sync_copy(zdeg_hbm.at[pl.ds(r0, RPT)], degd_sh.at[pl.ds(r0, RPT)])
    pltpu.sync_copy(ones_hbm, ones_v)
    pltpu.sync_copy(batch3.at[wid], bidx_v)
    pltpu.sync_copy(e4.at[0, wid], sidx_v)
    pltpu.sync_copy(e4.at[1, wid], didx_v)

    for c in range(NBCH):
        pltpu.async_copy(emb_hbm.at[bidx_v.at[c]],
                         erows_v.at[pl.ds(c * BCH, BCH)], gsem)
    for c in range(NBCH):
        pltpu.make_async_copy(emb_hbm.at[bidx_v.at[c]],
                              erows_v.at[pl.ds(c * BCH, BCH)], gsem).wait()
    pltpu.sync_copy(erows_v,
                    embs_out.at[pl.ds(wid * (NPAD // NW), NBCH * BCH)])
    plsc.subcore_barrier()

    def _dscat(c):
        pltpu.async_copy(ones_v, degs_sh.at[sidx_v.at[c]], dsem_s, add=True)
        pltpu.async_copy(ones_v, degd_sh.at[didx_v.at[c]], dsem_d, add=True)

    def _dscat_wait(c):
        pltpu.make_async_copy(ones_v, degs_sh.at[sidx_v.at[c]], dsem_s).wait()
        pltpu.make_async_copy(ones_v, degd_sh.at[didx_v.at[c]], dsem_d).wait()

    _dscat(0)

    def deg_chunk(c, carry):
        _dscat(c)
        _dscat_wait(c - 1)
        return carry

    lax.fori_loop(1, NECH, deg_chunk, 0)
    _dscat_wait(NECH - 1)
    plsc.subcore_barrier()
    pltpu.sync_copy(degs_sh.at[pl.ds(r0, RPT)],
                    degp_out.at[cid, 0, pl.ds(r0, RPT)])
    pltpu.sync_copy(degd_sh.at[pl.ds(r0, RPT)],
                    degp_out.at[cid, 1, pl.ds(r0, RPT)])


ECHM = 64
NECHM = EPAD // NW // ECHM
HECH = NECHM // 2
NBUF = 4


@functools.partial(
    pl.kernel,
    out_type=jax.ShapeDtypeStruct((NC, NPAD, HID), jnp.float32),
    mesh=_mesh,
    scratch_types=[
        pltpu.VMEM((HECH, ECHM), jnp.int32),
        pltpu.VMEM((HECH, ECHM), jnp.int32),
        [pltpu.VMEM((ECHM, HID), jnp.float32)] * NBUF,
        pltpu.VMEM_SHARED((NPAD, HID), jnp.float32),
        [pltpu.SemaphoreType.DMA] * NBUF,
        [pltpu.SemaphoreType.DMA] * NBUF,
    ],
    compiler_params=_sc_params,
)
def _msgpass(x_hbm, e5, zfeat_hbm,
             part_out,
             sidx_v, didx_v, bufs, acc_sh, gsems, ssems):
    cid = lax.axis_index("c")
    sid = lax.axis_index("s")
    wid = sid * NC + cid
    r0 = sid * RPT

    def _gather(c, j):
        return pltpu.async_copy(x_hbm.at[sidx_v.at[c]], bufs[j], gsems[j])

    def _scatter(c, j):
        return pltpu.async_copy(bufs[j], acc_sh.at[didx_v.at[c]],
                                ssems[j], add=True)

    def _gather_wait(c, j):
        pltpu.make_async_copy(x_hbm.at[sidx_v.at[c]], bufs[j],
                              gsems[j]).wait()

    def _scatter_wait(c, j):
        pltpu.make_async_copy(bufs[j], acc_sh.at[didx_v.at[c]],
                              ssems[j]).wait()

    pltpu.sync_copy(e5.at[0, wid, 0], sidx_v)
    pltpu.sync_copy(e5.at[1, wid, 0], didx_v)
    for j in range(NBUF):
        _gather(j, j)
    pltpu.sync_copy(zfeat_hbm.at[pl.ds(r0, RPT)], acc_sh.at[pl.ds(r0, RPT)])
    plsc.subcore_barrier()

    for h in range(2):
        if h:
            pltpu.sync_copy(e5.at[0, wid, h], sidx_v)
            pltpu.sync_copy(e5.at[1, wid, h], didx_v)
            for j in range(NBUF):
                _gather(j, j)

        def body(k, carry):
            base = NBUF * k
            for j in range(NBUF):
                _gather_wait(base + j, j)
                _scatter(base + j, j)
            for j in range(NBUF):
                _scatter_wait(base + j, j)
                _gather(base + NBUF + j, j)
            return carry

        lax.fori_loop(0, HECH // NBUF - 1, body, 0)
        base = HECH - NBUF
        for j in range(NBUF):
            _gather_wait(base + j, j)
            _scatter(base + j, j)
        for j in range(NBUF):
            _scatter_wait(base + j, j)
    plsc.subcore_barrier()
    pltpu.sync_copy(acc_sh.at[pl.ds(r0, RPT)],
                    part_out.at[cid, pl.ds(r0, RPT)])


RB = 2048


def _scale_body(x_ref, d0_ref, d1_ref, o_ref):
    deg = d0_ref[...] + d1_ref[...]
    o_ref[...] = x_ref[...] * lax.rsqrt(jnp.clip(deg, 1.0, None))


def _scale(x, d0, d1):
    return pl.pallas_call(
        _scale_body,
        grid=(NPAD // RB,),
        in_specs=[
            pl.BlockSpec((RB, HID), lambda i: (i, 0)),
            pl.BlockSpec((RB, 1), lambda i: (i, 0)),
            pl.BlockSpec((RB, 1), lambda i: (i, 0)),
        ],
        out_specs=pl.BlockSpec((RB, HID), lambda i: (i, 0)),
        out_shape=jax.ShapeDtypeStruct((NPAD, HID), jnp.float32),
    )(x, d0, d1)


def _layer_body(p_ref, dd0_ref, dd1_ref, ds0_ref, ds1_ref,
                w_ref, b_ref, o_ref, *, post_src):
    agg = p_ref[0] + p_ref[1]
    nd = lax.rsqrt(jnp.clip(dd0_ref[...] + dd1_ref[...], 1.0, None))
    y = jnp.dot(agg * nd, w_ref[...], preferred_element_type=jnp.float32)
    y = jnp.maximum(y + b_ref[...], 0.0)
    if post_src:
        y = y * lax.rsqrt(jnp.clip(ds0_ref[...] + ds1_ref[...], 1.0, None))
    o_ref[...] = y


def _layer(parts, dd0, dd1, ds0, ds1, w, b, post_src):
    col = pl.BlockSpec((RB, 1), lambda i: (i, 0))
    return pl.pallas_call(
        functools.partial(_layer_body, post_src=post_src),
        grid=(NPAD // RB,),
        in_specs=[
            pl.BlockSpec((NC, RB, HID), lambda i: (0, i, 0)),
            col, col, col, col,
            pl.BlockSpec((HID, HID), lambda i: (0, 0)),
            pl.BlockSpec((1, HID), lambda i: (0, 0)),
        ],
        out_specs=pl.BlockSpec((RB, HID), lambda i: (i, 0)),
        out_shape=jax.ShapeDtypeStruct((NPAD, HID), jnp.float32),
    )(parts, dd0, dd1, ds0, ds1, w, b)


def kernel(batch, edge_index, emb_table, W1, b1, W2, b2):
    pe = EPAD - E
    pad_rows = N + jnp.arange(pe, dtype=jnp.int32) % (NPAD - N)
    epad = jnp.concatenate(
        [edge_index, jnp.broadcast_to(pad_rows, (2, pe))], axis=1)
    e4 = epad.reshape(2, NW, NECH, ECH)
    e5 = epad.reshape(2, NW, 2, HECH, ECHM)
    pb = NPAD - N
    batch3 = jnp.concatenate(
        [batch, jnp.arange(pb, dtype=jnp.int32) * 131 % emb_table.shape[0]]
    ).reshape(NW, NBCH, BCH)
    ones_e = jnp.ones((ECH,), jnp.float32)
    zdeg = jnp.zeros((NPAD,), jnp.float32)
    zfeat = jnp.zeros((NPAD, HID), jnp.float32)

    embs, degp = _phase_a(emb_table, batch3, e4, ones_e, zdeg)
    ds0 = degp[0, 0].reshape(NPAD, 1)
    ds1 = degp[1, 0].reshape(NPAD, 1)
    dd0 = degp[0, 1].reshape(NPAD, 1)
    dd1 = degp[1, 1].reshape(NPAD, 1)
    b1r = b1.reshape(1, HID)
    b2r = b2.reshape(1, HID)

    h0 = _scale(embs, ds0, ds1)
    parts = _msgpass(h0, e5, zfeat)
    h1 = _layer(parts, dd0, dd1, ds0, ds1, W1, b1r, True)
    parts2 = _msgpass(h1, e5, zfeat)
    out = _layer(parts2, dd0, dd1, ds0, ds1, W2, b2r, False)
    return out[:N]
